# Initial kernel scaffold; baseline (speedup 1.0000x reference)
import jax
import jax.numpy as jnp
from jax.experimental import pallas as pl


def kernel(x, edge_index, W1_l, b1, W1_r, gamma, beta, W2_l, b2, W2_r):
    raise NotImplementedError("write your pallas kernel here")


# speedup vs baseline: 5.9313x; 5.9313x over previous
import functools

import jax
import jax.numpy as jnp
from jax import lax
from jax.experimental import pallas as pl
from jax.experimental.pallas import tpu as pltpu
from jax.experimental.pallas import tpu_sc as plsc

N_NODES_ = 10000
N_EDGES_ = 160000
_NT = 16
_EB = 125
_EPT = N_EDGES_ // _NT
_NB = _EPT // _EB
_NPAD = 10240
_STR = _NPAD // _NT


def _make_sc_scatter(nch, with_deg):
    ch_per_core = nch // 2
    mesh = plsc.VectorSubcoreMesh(core_axis_name="c", subcore_axis_name="s")

    out_type = [jax.ShapeDtypeStruct((nch, _NPAD, 128), jnp.float32)]
    if with_deg:
        out_type.append(jax.ShapeDtypeStruct((_NPAD, 128), jnp.float32))

    scratch = [
        pltpu.VMEM((_NB, _EB), jnp.int32),
        pltpu.VMEM((_NB, _EB), jnp.int32),
        pltpu.VMEM((_EB, 128), jnp.float32),
        pltpu.SemaphoreType.DMA,
        pltpu.VMEM_SHARED((_NPAD, 128), jnp.float32),
    ]

    def body(y3, src2, dst2, zer128, ones128, *refs):
        if with_deg:
            out_agg, out_deg, src_v, dst_v, gbuf, sem, agg_sh = refs
        else:
            out_agg, src_v, dst_v, gbuf, sem, agg_sh = refs
        c = lax.axis_index("c")
        s = lax.axis_index("s")
        stripe = pl.ds(s * _STR, _STR)

        pltpu.sync_copy(src2.at[s], src_v)
        pltpu.sync_copy(dst2.at[s], dst_v)

        npass = nch + 1 if with_deg else nch
        for ci in range(npass):
            is_deg = ci == nch
            owner = 0 if is_deg else ci // ch_per_core

            @pl.when(c == owner)
            def _(ci=ci, is_deg=is_deg):
                pltpu.sync_copy(zer128, agg_sh.at[stripe])
                if is_deg:
                    pltpu.sync_copy(ones128, gbuf)
                plsc.subcore_barrier()

                def batch(j, carry):
                    if is_deg:
                        pltpu.sync_copy(gbuf, agg_sh.at[dst_v.at[j]],
                                        add=True)
                    else:
                        pltpu.async_copy(y3.at[ci].at[src_v.at[j]], gbuf,
                                         sem).wait()
                        pltpu.sync_copy(gbuf, agg_sh.at[dst_v.at[j]],
                                        add=True)
                    return carry

                lax.fori_loop(0, _NB, batch, 0)
                plsc.subcore_barrier()
                dst_ref = out_deg if is_deg else out_agg.at[ci]
                pltpu.sync_copy(agg_sh.at[stripe], dst_ref.at[stripe])

    return pl.kernel(body, out_type=out_type, mesh=mesh,
                     scratch_types=scratch)


_sc_scatter_deg = _make_sc_scatter(8, True)
_sc_scatter = _make_sc_scatter(4, False)


def _sc_segment_sum(y, src2, dst2, with_deg):
    n, d = y.shape
    nch = d // 128
    y3 = y.reshape(n, nch, 128).transpose(1, 0, 2)
    zer128 = jnp.zeros((_STR, 128), jnp.float32)
    ones128 = jnp.ones((_EB, 128), jnp.float32)
    if with_deg:
        aggc, deg = _sc_scatter_deg(y3, src2, dst2, zer128, ones128)
        deg = deg[:n]
    else:
        (aggc,) = _sc_scatter(y3, src2, dst2, zer128, ones128)
        deg = None
    agg = aggc[:, :n].transpose(1, 0, 2).reshape(n, d)
    return agg, deg


def _mm_body(x_ref, w_ref, o_ref):
    o_ref[...] = jnp.dot(x_ref[...], w_ref[...],
                         preferred_element_type=jnp.float32)


def _matmul(x, w, bm, bn):
    m, k = x.shape
    _, n = w.shape
    return pl.pallas_call(
        _mm_body,
        grid=(n // bn, m // bm),
        in_specs=[pl.BlockSpec((bm, k), lambda j, i: (i, 0)),
                  pl.BlockSpec((k, bn), lambda j, i: (0, j))],
        out_specs=pl.BlockSpec((bm, bn), lambda j, i: (i, j)),
        out_shape=jax.ShapeDtypeStruct((m, n), jnp.float32),
    )(x, w)


def _hpre_body(agg_ref, z_ref, deg_ref, b_ref, h_ref, st_ref):
    i = pl.program_id(0)

    @pl.when(i == 0)
    def _():
        st_ref[...] = jnp.zeros_like(st_ref)

    d = jnp.maximum(deg_ref[...][:, 0:1], 1.0)
    h = jnp.maximum(agg_ref[...] / d + b_ref[...] + z_ref[...], 0.0)
    h_ref[...] = h
    st_ref[0:1, :] += jnp.sum(h, axis=0, keepdims=True)
    st_ref[1:2, :] += jnp.sum(h * h, axis=0, keepdims=True)


def _hpre(agg, z, deg, b, bm):
    m, n = agg.shape
    return pl.pallas_call(
        _hpre_body,
        grid=(m // bm,),
        in_specs=[pl.BlockSpec((bm, n), lambda i: (i, 0)),
                  pl.BlockSpec((bm, n), lambda i: (i, 0)),
                  pl.BlockSpec((bm, 128), lambda i: (i, 0)),
                  pl.BlockSpec((1, n), lambda i: (0, 0))],
        out_specs=[pl.BlockSpec((bm, n), lambda i: (i, 0)),
                   pl.BlockSpec((8, n), lambda i: (0, 0))],
        out_shape=[jax.ShapeDtypeStruct((m, n), jnp.float32),
                   jax.ShapeDtypeStruct((8, n), jnp.float32)],
    )(agg, z, deg, b)


def _bn_mm_body(h_ref, st_ref, g_ref, be_ref, w_ref, o_ref):
    inv_n = 1.0 / N_NODES_
    mu = st_ref[0:1, :] * inv_n
    var = st_ref[1:2, :] * inv_n - mu * mu
    a = g_ref[...] * lax.rsqrt(var + 1e-5)
    c = be_ref[...] - mu * a
    hb = h_ref[...] * a + c
    o_ref[...] = jnp.dot(hb, w_ref[...], preferred_element_type=jnp.float32)


def _bn_mm(h, st, gamma, beta, w, bm, bn):
    m, k = h.shape
    _, n = w.shape
    return pl.pallas_call(
        _bn_mm_body,
        grid=(n // bn, m // bm),
        in_specs=[pl.BlockSpec((bm, k), lambda j, i: (i, 0)),
                  pl.BlockSpec((8, k), lambda j, i: (0, 0)),
                  pl.BlockSpec((1, k), lambda j, i: (0, 0)),
                  pl.BlockSpec((1, k), lambda j, i: (0, 0)),
                  pl.BlockSpec((k, bn), lambda j, i: (0, j))],
        out_specs=pl.BlockSpec((bm, bn), lambda j, i: (i, j)),
        out_shape=jax.ShapeDtypeStruct((m, n), jnp.float32),
    )(h, st, gamma, beta, w)


def _final_body(agg_ref, z_ref, deg_ref, b_ref, o_ref):
    d = jnp.maximum(deg_ref[...][:, 0:1], 1.0)
    o_ref[...] = agg_ref[...] / d + b_ref[...] + z_ref[...]


def _final(agg, z, deg, b, bm):
    m, n = agg.shape
    return pl.pallas_call(
        _final_body,
        grid=(m // bm,),
        in_specs=[pl.BlockSpec((bm, n), lambda i: (i, 0)),
                  pl.BlockSpec((bm, n), lambda i: (i, 0)),
                  pl.BlockSpec((bm, 128), lambda i: (i, 0)),
                  pl.BlockSpec((1, n), lambda i: (0, 0))],
        out_specs=pl.BlockSpec((bm, n), lambda i: (i, 0)),
        out_shape=jax.ShapeDtypeStruct((m, n), jnp.float32),
    )(agg, z, deg, b)


@jax.jit
def kernel(x, edge_index, W1_l, b1, W1_r, gamma, beta, W2_l, b2, W2_r):
    src2 = edge_index[0].reshape(_NT, _NB, _EB)
    dst2 = edge_index[1].reshape(_NT, _NB, _EB)

    w1 = jnp.concatenate([W1_l, W1_r], axis=1)
    y1z1 = _matmul(x, w1, 400, 1024)
    y1 = y1z1[:, :1024]
    z1 = y1z1[:, 1024:]

    agg1, deg = _sc_segment_sum(y1, src2, dst2, True)
    h, st = _hpre(agg1, z1, deg, b1.reshape(1, -1), 1000)

    w2 = jnp.concatenate([W2_l, W2_r], axis=1)
    y2z2 = _bn_mm(h, st, gamma.reshape(1, -1), beta.reshape(1, -1),
                  w2, 1000, 512)
    y2 = y2z2[:, :512]
    z2 = y2z2[:, 512:]

    agg2, _ = _sc_segment_sum(y2, src2, dst2, False)
    return _final(agg2, z2, deg, b2.reshape(1, -1), 1000)
